# baseline (speedup 1.0000x reference)
import functools

import jax
import jax.numpy as jnp
from jax import lax
from jax.experimental import pallas as pl
from jax.experimental.pallas import tpu as pltpu
from jax.experimental.pallas import tpu_sc as plsc

NUM_CLS = 1000000
EMB_DIM = 16
BATCH = 16384
HIST = 200

NC = 2
NS = 16
NW = NC * NS
BW = BATCH // NW
HCH = 2
NCHUNK = HIST // HCH
NBUF = 2
LANES = 16

_mesh = plsc.VectorSubcoreMesh(core_axis_name="c", subcore_axis_name="s")

TCW = 1600
TNCH = NUM_CLS // TCW


@functools.partial(
    pl.kernel,
    out_type=jax.ShapeDtypeStruct((NUM_CLS, EMB_DIM), jnp.float32),
    mesh=_mesh,
    scratch_types=[
        pltpu.VMEM((EMB_DIM, TCW), jnp.float32),
        pltpu.VMEM((EMB_DIM, TCW), jnp.float32),
        pltpu.VMEM((TCW, EMB_DIM), jnp.float32),
        pltpu.VMEM((TCW, EMB_DIM), jnp.float32),
        pltpu.SemaphoreType.DMA,
        pltpu.SemaphoreType.DMA,
        pltpu.SemaphoreType.DMA,
        pltpu.SemaphoreType.DMA,
    ],
    compiler_params=pltpu.CompilerParams(
        use_tc_tiling_on_sc=False, needs_layout_passes=False),
)
def _sc_table_transpose(embt_hbm, out_hbm,
                        in_v0, in_v1, to_v0, to_v1,
                        s_i0, s_i1, s_o0, s_o1):
    wid = lax.axis_index("s") * NC + lax.axis_index("c")
    in_bufs = (in_v0, in_v1)
    to_bufs = (to_v0, to_v1)
    s_i = (s_i0, s_i1)
    s_o = (s_o0, s_o1)
    lane_iota = lax.iota(jnp.int32, LANES)
    e_splats = [jnp.full((LANES,), e, jnp.int32) for e in range(EMB_DIM)]
    nloc = (TNCH - wid + NW - 1) // NW

    def start_in(c, b):
        i0 = (c * NW + wid) * TCW
        pltpu.async_copy(
            embt_hbm.at[:, pl.ds(i0, TCW)], in_bufs[b], s_i[b])

    def wait_in(b):
        pltpu.make_async_copy(
            embt_hbm.at[:, pl.ds(0, TCW)], in_bufs[b], s_i[b]).wait()

    def wait_to(b):
        pltpu.make_async_copy(
            to_bufs[b], out_hbm.at[pl.ds(0, TCW)], s_o[b]).wait()

    @pl.when(nloc > 0)
    def _():
        start_in(0, 0)

    @pl.when(nloc > 1)
    def _():
        start_in(1, 1)

    def tloop(t, carry):
        for b in range(NBUF):
            c = t * NBUF + b

            @pl.when(c < nloc)
            def _():
                wait_in(b)

                @pl.when(c >= NBUF)
                def _():
                    wait_to(b)

                @plsc.parallel_loop(0, TCW, 1, unroll=8)
                def _(j):
                    j_splat = jnp.full((LANES,), 0, jnp.int32) + j
                    vec = plsc.load_gather(in_bufs[b], [lane_iota, j_splat])
                    to_bufs[b][j] = vec

                i0 = (c * NW + wid) * TCW
                pltpu.async_copy(
                    to_bufs[b], out_hbm.at[pl.ds(i0, TCW)], s_o[b])

                @pl.when(c + NBUF < nloc)
                def _():
                    start_in(c + NBUF, b)
        return carry

    lax.fori_loop(0, (TNCH + NW - 1) // NW // NBUF + 1, tloop, 0)

    @pl.when(nloc > 0)
    def _():
        wait_to(0)

    @pl.when(nloc > 1)
    def _():
        wait_to(1)


@functools.partial(
    pl.kernel,
    out_type=jax.ShapeDtypeStruct(
        (HIST, EMB_DIM // 8, BATCH // 128, 8, 128), jnp.float32),
    mesh=_mesh,
    scratch_types=[
        pltpu.VMEM((HCH, BW), jnp.int32),
        pltpu.VMEM((HCH, BW), jnp.int32),
        pltpu.VMEM((HCH * BW, EMB_DIM), jnp.float32),
        pltpu.VMEM((HCH * BW, EMB_DIM), jnp.float32),
        pltpu.VMEM((HCH, EMB_DIM // 8, BW // 128, 8, 128), jnp.float32),
        pltpu.VMEM((HCH, EMB_DIM // 8, BW // 128, 8, 128), jnp.float32),
        pltpu.SemaphoreType.DMA,
        pltpu.SemaphoreType.DMA,
        pltpu.SemaphoreType.DMA,
        pltpu.SemaphoreType.DMA,
        pltpu.SemaphoreType.DMA,
        pltpu.SemaphoreType.DMA,
    ],
    compiler_params=pltpu.CompilerParams(
        use_tc_tiling_on_sc=False, needs_layout_passes=False),
)
def _sc_gather(idx_hbm, emb_hbm, out_hbm,
               idx_v0, idx_v1, rows_v0, rows_v1, trows_v0, trows_v1,
               s_i0, s_i1, s_g0, s_g1, s_o0, s_o1):
    wid = lax.axis_index("s") * NC + lax.axis_index("c")
    b0 = pl.multiple_of(wid * BW, BW)
    idx_bufs = (idx_v0, idx_v1)
    row_bufs = (rows_v0, rows_v1)
    trow_bufs = (trows_v0, trows_v1)
    s_i = (s_i0, s_i1)
    s_g = (s_g0, s_g1)
    s_o = (s_o0, s_o1)

    lane_iota = lax.iota(jnp.int32, LANES)
    e_splats = [jnp.full((LANES,), e, jnp.int32) for e in range(EMB_DIM)]

    def start_idx(g, b):
        h0 = pl.multiple_of(g * HCH, HCH)
        pltpu.async_copy(
            idx_hbm.at[pl.ds(h0, HCH), pl.ds(b0, BW)], idx_bufs[b], s_i[b])

    def wait_idx(b):
        pltpu.make_async_copy(
            idx_hbm.at[pl.ds(0, HCH), pl.ds(0, BW)], idx_bufs[b], s_i[b]).wait()

    def wait_out(b):
        pltpu.make_async_copy(
            trow_bufs[b],
            out_hbm.at[pl.ds(0, HCH), :, pl.ds(0, BW // 128)],
            s_o[b]).wait()

    def fire_gathers(b):
        for hh in range(HCH):
            pltpu.async_copy(
                emb_hbm.at[idx_bufs[b].at[hh]],
                row_bufs[b].at[pl.ds(hh * BW, BW)],
                s_g[b],
            )

    def drain_gathers(b):
        for hh in range(HCH):
            pltpu.make_async_copy(
                emb_hbm.at[idx_bufs[b].at[hh]],
                row_bufs[b].at[pl.ds(hh * BW, BW)],
                s_g[b],
            ).wait()

    def transpose(b):
        @plsc.parallel_loop(0, BW // LANES, 1, unroll=4)
        def _(i):
            base = pl.multiple_of(i * LANES, LANES)
            bt = i // 8
            bg = pl.multiple_of((i % 8) * LANES, LANES)
            for hh in range(HCH):
                row_idx = hh * BW + base + lane_iota
                for e in range(EMB_DIM):
                    vec = plsc.load_gather(
                        row_bufs[b], [row_idx, e_splats[e]])
                    trow_bufs[b][hh, e // 8, bt, e % 8,
                                 pl.ds(bg, LANES)] = vec

    def start_out(g, b):
        h0 = pl.multiple_of(g * HCH, HCH)
        pltpu.async_copy(
            trow_bufs[b],
            out_hbm.at[pl.ds(h0, HCH), :,
                       pl.ds(wid * (BW // 128), BW // 128)],
            s_o[b])

    start_idx(0, 0)
    start_idx(1, 1)
    wait_idx(0)
    fire_gathers(0)

    def outer(t, carry):
        for b in range(NBUF):
            g = t * NBUF + b
            bn = 1 - b

            @pl.when(g + 1 < NCHUNK)
            def _():
                wait_idx(bn)
                fire_gathers(bn)

            drain_gathers(b)

            @pl.when(t > 0)
            def _():
                wait_out(b)

            transpose(b)
            start_out(g, b)

            @pl.when(g + NBUF < NCHUNK)
            def _():
                start_idx(g + NBUF, b)
        return carry

    lax.fori_loop(0, NCHUNK // NBUF, outer, 0)
    wait_out(0)
    wait_out(1)


def kernel(idx, emb):
    idx_t = jnp.swapaxes(idx.astype(jnp.int32), 0, 1)
    emb_rm = _sc_table_transpose(jnp.swapaxes(emb, 0, 1))
    out5 = _sc_gather(idx_t, emb_rm)
    return jnp.transpose(out5, (2, 4, 0, 1, 3)).reshape(BATCH, HIST, EMB_DIM)


# speedup vs baseline: 2.0796x; 2.0796x over previous
import functools

import jax
import jax.numpy as jnp
from jax import lax
from jax.experimental import pallas as pl
from jax.experimental.pallas import tpu as pltpu
from jax.experimental.pallas import tpu_sc as plsc

NUM_CLS = 1000000
EMB_DIM = 16
BATCH = 16384
HIST = 200

NC = 2
NS = 16
NW = NC * NS
BW = BATCH // NW
HCH = 2
NCHUNK = HIST // HCH
NBUF = 2
LANES = 16

_mesh = plsc.VectorSubcoreMesh(core_axis_name="c", subcore_axis_name="s")


@functools.partial(
    pl.kernel,
    out_type=jax.ShapeDtypeStruct(
        (HIST, EMB_DIM // 8, BATCH // 128, 8, 128), jnp.float32),
    mesh=_mesh,
    scratch_types=[
        pltpu.VMEM((HCH, BW), jnp.int32),
        pltpu.VMEM((HCH, BW), jnp.int32),
        pltpu.VMEM((HCH * BW, EMB_DIM), jnp.float32),
        pltpu.VMEM((HCH * BW, EMB_DIM), jnp.float32),
        pltpu.VMEM((HCH, EMB_DIM // 8, BW // 128, 8, 128), jnp.float32),
        pltpu.VMEM((HCH, EMB_DIM // 8, BW // 128, 8, 128), jnp.float32),
        pltpu.SemaphoreType.DMA,
        pltpu.SemaphoreType.DMA,
        pltpu.SemaphoreType.DMA,
        pltpu.SemaphoreType.DMA,
        pltpu.SemaphoreType.DMA,
        pltpu.SemaphoreType.DMA,
    ],
    compiler_params=pltpu.CompilerParams(
        use_tc_tiling_on_sc=False, needs_layout_passes=False),
)
def _sc_gather(idx_hbm, emb_hbm, out_hbm,
               idx_v0, idx_v1, rows_v0, rows_v1, trows_v0, trows_v1,
               s_i0, s_i1, s_g0, s_g1, s_o0, s_o1):
    wid = lax.axis_index("s") * NC + lax.axis_index("c")
    b0 = pl.multiple_of(wid * BW, BW)
    idx_bufs = (idx_v0, idx_v1)
    row_bufs = (rows_v0, rows_v1)
    trow_bufs = (trows_v0, trows_v1)
    s_i = (s_i0, s_i1)
    s_g = (s_g0, s_g1)
    s_o = (s_o0, s_o1)

    lane_iota = lax.iota(jnp.int32, LANES)
    e_splats = [jnp.full((LANES,), e, jnp.int32) for e in range(EMB_DIM)]

    def start_idx(g, b):
        h0 = pl.multiple_of(g * HCH, HCH)
        pltpu.async_copy(
            idx_hbm.at[pl.ds(h0, HCH), pl.ds(b0, BW)], idx_bufs[b], s_i[b])

    def wait_idx(b):
        pltpu.make_async_copy(
            idx_hbm.at[pl.ds(0, HCH), pl.ds(0, BW)], idx_bufs[b], s_i[b]).wait()

    def wait_out(b):
        pltpu.make_async_copy(
            trow_bufs[b],
            out_hbm.at[pl.ds(0, HCH), :, pl.ds(0, BW // 128)],
            s_o[b]).wait()

    def fire_gathers(b):
        for hh in range(HCH):
            pltpu.async_copy(
                emb_hbm.at[idx_bufs[b].at[hh]],
                row_bufs[b].at[pl.ds(hh * BW, BW)],
                s_g[b],
            )

    def drain_gathers(b):
        for hh in range(HCH):
            pltpu.make_async_copy(
                emb_hbm.at[idx_bufs[b].at[hh]],
                row_bufs[b].at[pl.ds(hh * BW, BW)],
                s_g[b],
            ).wait()

    def transpose(b):
        @plsc.parallel_loop(0, BW // LANES, 1, unroll=4)
        def _(i):
            base = pl.multiple_of(i * LANES, LANES)
            bt = i // 8
            bg = pl.multiple_of((i % 8) * LANES, LANES)
            for hh in range(HCH):
                row_idx = hh * BW + base + lane_iota
                for e in range(EMB_DIM):
                    vec = plsc.load_gather(
                        row_bufs[b], [row_idx, e_splats[e]])
                    trow_bufs[b][hh, e // 8, bt, e % 8,
                                 pl.ds(bg, LANES)] = vec

    def start_out(g, b):
        h0 = pl.multiple_of(g * HCH, HCH)
        pltpu.async_copy(
            trow_bufs[b],
            out_hbm.at[pl.ds(h0, HCH), :,
                       pl.ds(wid * (BW // 128), BW // 128)],
            s_o[b])

    start_idx(0, 0)
    start_idx(1, 1)
    wait_idx(0)
    fire_gathers(0)

    def outer(t, carry):
        for b in range(NBUF):
            g = t * NBUF + b
            bn = 1 - b

            @pl.when(g + 1 < NCHUNK)
            def _():
                wait_idx(bn)
                fire_gathers(bn)

            drain_gathers(b)

            @pl.when(t > 0)
            def _():
                wait_out(b)

            transpose(b)
            start_out(g, b)

            @pl.when(g + NBUF < NCHUNK)
            def _():
                start_idx(g + NBUF, b)
        return carry

    lax.fori_loop(0, NCHUNK // NBUF, outer, 0)
    wait_out(0)
    wait_out(1)


def kernel(idx, emb):
    idx_t = jnp.swapaxes(idx.astype(jnp.int32), 0, 1)
    out5 = _sc_gather(idx_t, emb)
    return jnp.transpose(out5, (2, 4, 0, 1, 3)).reshape(BATCH, HIST, EMB_DIM)
